# baseline (speedup 1.0000x reference)
import functools

import jax
import jax.numpy as jnp
from jax import lax
from jax.experimental import pallas as pl
from jax.experimental.pallas import tpu as pltpu
from jax.experimental.pallas import tpu_sc as plsc

N_B = 16384
D = 64
REG = 0.01
NC, NS = 2, 16
NW = NC * NS
ROWS_W = N_B // NW
CHUNK = 128
NCH = ROWS_W // CHUNK


def _sc_gather_dot(u2, i2, j2, u_emb, i_emb):
    mesh = plsc.VectorSubcoreMesh(core_axis_name="c", subcore_axis_name="s")

    @functools.partial(
        pl.kernel,
        out_type=(
            jax.ShapeDtypeStruct((N_B,), jnp.float32),
            jax.ShapeDtypeStruct((NW, 16), jnp.float32),
        ),
        mesh=mesh,
        compiler_params=pltpu.CompilerParams(
            needs_layout_passes=False, use_tc_tiling_on_sc=False),
        scratch_types=(
            pltpu.VMEM((NCH, CHUNK), jnp.int32),
            pltpu.VMEM((NCH, CHUNK), jnp.int32),
            pltpu.VMEM((NCH, CHUNK), jnp.int32),
            pltpu.VMEM((ROWS_W, D), jnp.float32),
            pltpu.VMEM((ROWS_W, D), jnp.float32),
            pltpu.VMEM((ROWS_W, D), jnp.float32),
            pltpu.VMEM((ROWS_W,), jnp.float32),
            pltpu.VMEM((16,), jnp.float32),
            pltpu.SemaphoreType.DMA,
        ),
    )
    def k(u_hbm, i_hbm, j_hbm, ue_hbm, ie_hbm, d_out, sq_out,
          uix, iix, jix, ur, pr, nr, dv, sqv, sem):
        w = lax.axis_index("s") * NC + lax.axis_index("c")
        pltpu.sync_copy(u_hbm.at[pl.ds(w * NCH, NCH)], uix)
        pltpu.sync_copy(i_hbm.at[pl.ds(w * NCH, NCH)], iix)
        pltpu.sync_copy(j_hbm.at[pl.ds(w * NCH, NCH)], jix)
        handles = []
        for c in range(NCH):
            rows = pl.ds(c * CHUNK, CHUNK)
            handles.append(pltpu.async_copy(ue_hbm.at[uix.at[c]], ur.at[rows], sem))
            handles.append(pltpu.async_copy(ie_hbm.at[iix.at[c]], pr.at[rows], sem))
            handles.append(pltpu.async_copy(ie_hbm.at[jix.at[c]], nr.at[rows], sem))
        for h in handles:
            h.wait()

        lanes = lax.iota(jnp.int32, 16)

        def group(g, acc):
            rows = g * 16 + lanes
            dvec = jnp.zeros((16,), jnp.float32)
            for kk in range(D):
                kv = jnp.full((16,), kk, jnp.int32)
                uv = plsc.load_gather(ur, [rows, kv])
                pv = plsc.load_gather(pr, [rows, kv])
                nv = plsc.load_gather(nr, [rows, kv])
                dvec = dvec + uv * (pv - nv)
                acc = acc + uv * uv + pv * pv + nv * nv
            dv[pl.ds(g * 16, 16)] = dvec
            return acc

        acc = lax.fori_loop(0, ROWS_W // 16, group,
                            jnp.zeros((16,), jnp.float32))
        sqv[...] = acc
        pltpu.sync_copy(dv, d_out.at[pl.ds(w * ROWS_W, ROWS_W)])
        pltpu.sync_copy(sqv, sq_out.at[w])

    return k(u2, i2, j2, u_emb, i_emb)


def _tc_loss(d2, sq2):
    def body(d_ref, sq_ref, o_ref):
        d = d_ref[...]
        log_sig = jnp.log(jax.nn.sigmoid(d))
        sq = jnp.sum(sq_ref[...])
        o_ref[0, 0] = -(jnp.sum(log_sig) / N_B) + REG * (0.5 * sq / N_B)

    return pl.pallas_call(
        body,
        out_shape=jax.ShapeDtypeStruct((1, 1), jnp.float32),
        in_specs=[
            pl.BlockSpec(memory_space=pltpu.VMEM),
            pl.BlockSpec(memory_space=pltpu.VMEM),
        ],
        out_specs=pl.BlockSpec(memory_space=pltpu.SMEM),
    )(d2, sq2)


def kernel(u, i, j, u_g_embeddings, i_g_embeddings):
    u2 = u.astype(jnp.int32).reshape(N_B // CHUNK, CHUNK)
    i2 = i.astype(jnp.int32).reshape(N_B // CHUNK, CHUNK)
    j2 = j.astype(jnp.int32).reshape(N_B // CHUNK, CHUNK)
    d, sq = _sc_gather_dot(u2, i2, j2, u_g_embeddings, i_g_embeddings)
    out = _tc_loss(d.reshape(128, 128), sq.reshape(4, 128))
    return out.reshape(())


# speedup vs baseline: 1.5577x; 1.5577x over previous
import functools

import jax
import jax.numpy as jnp
from jax import lax
from jax.experimental import pallas as pl
from jax.experimental.pallas import tpu as pltpu
from jax.experimental.pallas import tpu_sc as plsc

N_B = 16384
D = 64
REG = 0.01
NC, NS = 2, 16
NW = NC * NS
ROWS_W = N_B // NW
HALF = ROWS_W // 2
NGH = HALF // 16


def _sc_gather_dot(u3, i3, j3, u_emb, i_emb):
    mesh = plsc.VectorSubcoreMesh(core_axis_name="c", subcore_axis_name="s")

    @functools.partial(
        pl.kernel,
        out_type=(
            jax.ShapeDtypeStruct((N_B,), jnp.float32),
            jax.ShapeDtypeStruct((NW, 16), jnp.float32),
        ),
        mesh=mesh,
        compiler_params=pltpu.CompilerParams(needs_layout_passes=False),
        scratch_types=(
            pltpu.VMEM((ROWS_W // 128, 128), jnp.int32),
            pltpu.VMEM((ROWS_W // 128, 128), jnp.int32),
            pltpu.VMEM((ROWS_W // 128, 128), jnp.int32),
            pltpu.VMEM((HALF, D), jnp.float32),
            pltpu.VMEM((HALF, D), jnp.float32),
            pltpu.VMEM((HALF, D), jnp.float32),
            pltpu.VMEM((ROWS_W,), jnp.float32),
            pltpu.VMEM((16,), jnp.float32),
            pltpu.SemaphoreType.DMA,
        ),
    )
    def k(u_hbm, i_hbm, j_hbm, ue_hbm, ie_hbm, d_out, sq_out,
          uix, iix, jix, ur, pr, nr, dv, sqv, sem):
        w = lax.axis_index("s") * NC + lax.axis_index("c")
        pltpu.sync_copy(u_hbm.at[w], uix)
        pltpu.sync_copy(i_hbm.at[w], iix)
        pltpu.sync_copy(j_hbm.at[w], jix)

        lanes = lax.iota(jnp.int32, 16)
        acc = jnp.zeros((16,), jnp.float32)

        for half in range(2):
            def fire(ix, tbl, dst):
                def grp(g, _):
                    vec = ix[half * 2 + g // 8, pl.ds((g % 8) * 16, 16)]
                    for rr in range(16):
                        pltpu.async_copy(tbl.at[vec[rr]],
                                         dst.at[g * 16 + rr], sem)
                    return 0
                lax.fori_loop(0, NGH, grp, 0)

            fire(uix, ue_hbm, ur)
            fire(iix, ie_hbm, pr)
            fire(jix, ie_hbm, nr)
            pltpu.make_async_copy(ue_hbm.at[pl.ds(0, HALF)], ur, sem).wait()
            pltpu.make_async_copy(ie_hbm.at[pl.ds(0, HALF)], pr, sem).wait()
            pltpu.make_async_copy(ie_hbm.at[pl.ds(0, HALF)], nr, sem).wait()

            def group(g, acc):
                rows = g * 16 + lanes
                dvec = jnp.zeros((16,), jnp.float32)
                for kk in range(D):
                    kv = jnp.full((16,), kk, jnp.int32)
                    uv = plsc.load_gather(ur, [rows, kv])
                    pv = plsc.load_gather(pr, [rows, kv])
                    nv = plsc.load_gather(nr, [rows, kv])
                    dvec = dvec + uv * (pv - nv)
                    acc = acc + uv * uv + pv * pv + nv * nv
                dv[pl.ds(half * HALF + g * 16, 16)] = dvec
                return acc

            acc = lax.fori_loop(0, NGH, group, acc)

        sqv[...] = acc
        pltpu.sync_copy(dv, d_out.at[pl.ds(w * ROWS_W, ROWS_W)])
        pltpu.sync_copy(sqv, sq_out.at[w])

    return k(u3, i3, j3, u_emb, i_emb)


def _tc_loss(d2, sq2):
    def body(d_ref, sq_ref, o_ref):
        d = d_ref[...]
        log_sig = jnp.log(jax.nn.sigmoid(d))
        sq = jnp.sum(sq_ref[...])
        o_ref[0, 0] = -(jnp.sum(log_sig) / N_B) + REG * (0.5 * sq / N_B)

    return pl.pallas_call(
        body,
        out_shape=jax.ShapeDtypeStruct((1, 1), jnp.float32),
        in_specs=[
            pl.BlockSpec(memory_space=pltpu.VMEM),
            pl.BlockSpec(memory_space=pltpu.VMEM),
        ],
        out_specs=pl.BlockSpec(memory_space=pltpu.SMEM),
    )(d2, sq2)


def kernel(u, i, j, u_g_embeddings, i_g_embeddings):
    u3 = u.astype(jnp.int32).reshape(NW, ROWS_W // 128, 128)
    i3 = i.astype(jnp.int32).reshape(NW, ROWS_W // 128, 128)
    j3 = j.astype(jnp.int32).reshape(NW, ROWS_W // 128, 128)
    d, sq = _sc_gather_dot(u3, i3, j3, u_g_embeddings, i_g_embeddings)
    out = _tc_loss(d.reshape(128, 128), sq.reshape(4, 128))
    return out.reshape(())
